# baseline (speedup 1.0000x reference)
import jax
import jax.numpy as jnp
from jax.experimental import pallas as pl
from jax.experimental.pallas import tpu as pltpu

ND = 4
D = 1024
B = 1024
S = 20
G = 32
PAD = B + ND * G
NBLK = PAD // G


def _route(ids):
    order = jnp.argsort(ids, stable=True).astype(jnp.int32)
    onehot = (ids[:, None] == jnp.arange(ND, dtype=ids.dtype)[None, :]).astype(jnp.int32)
    counts = jnp.sum(onehot, axis=0)
    rstarts = jnp.cumsum(counts) - counts
    padded = ((counts + G - 1) // G) * G
    ends = jnp.cumsum(padded)
    starts = ends - padded

    gs = jnp.arange(NBLK, dtype=jnp.int32) * G
    bdr = jnp.minimum(
        jnp.sum((gs[:, None] >= ends[None, :]).astype(jnp.int32), axis=1), ND - 1)
    d0 = ids[order[0]].astype(jnp.int32)
    bd = jnp.where(gs < ends[ND - 1], bdr, d0)
    return order, starts, rstarts, counts, bd


def _fused_body(order_ref, st_ref, rst_ref, cnt_ref, bd_ref,
                feat_ref, w_ref, a_ref, out_ref, l_ref,
                xacc, racc, wb, bsave, insem, outsem):
    g = pl.program_id(0)

    def issue_in(gg):
        pp = jax.lax.rem(gg, 2)
        bdv = bd_ref[gg]
        base = gg * G - st_ref[bdv]
        cntv = cnt_ref[bdv]
        rstv = rst_ref[bdv]
        for t in range(G):
            q = base + t
            b = order_ref[rstv + jnp.where(q < cntv, q, 0)]
            bsave[pp, t] = b
            pltpu.make_async_copy(
                feat_ref.at[:, b, :], xacc.at[pp, t], insem.at[pp, t]
            ).start()

    def wait_in(pp):
        for t in range(G):
            pltpu.make_async_copy(
                feat_ref.at[:, 0, :], xacc.at[pp, t], insem.at[pp, t]
            ).wait()

    def issue_out(gg):
        pp = jax.lax.rem(gg, 2)
        for t in range(G):
            b = bsave[pp, t]
            pltpu.make_async_copy(
                racc.at[pp, t], out_ref.at[:, b, :], outsem.at[pp, t]
            ).start()

    def wait_out(pp):
        for t in range(G):
            pltpu.make_async_copy(
                racc.at[pp, t], out_ref.at[:, 0, :], outsem.at[pp, t]
            ).wait()

    @pl.when(g == 0)
    def _():
        issue_in(jnp.int32(0))

    @pl.when(g + 1 < NBLK)
    def _():
        issue_in(g + 1)

    @pl.when(g >= 2)
    def _():
        wait_out(jax.lax.rem(g, 2))

    @pl.when(g < NBLK)
    def _():
        p = jax.lax.rem(g, 2)
        wait_in(p)

        @pl.when((g == 0) | (bd_ref[jnp.maximum(g - 1, 0)] != bd_ref[g]))
        def _():
            wb[...] = w_ref[0].astype(jnp.bfloat16)

        x = xacc[p].astype(jnp.bfloat16)
        w = wb[...]
        res = jax.lax.dot_general(x, w, dimension_numbers=(((2,), (1,)), ((), ())),
                                  preferred_element_type=jnp.float32)
        racc[p] = res
        issue_out(g)

    @pl.when((g >= 1) & (g <= NBLK))
    def _():
        pm = jax.lax.rem(g + 1, 2)
        gm = g - 1
        bdv = bd_ref[gm]
        nvalid = jnp.clip(cnt_ref[bdv] - (gm * G - st_ref[bdv]), 0, G)
        diff = racc[pm] - a_ref[...]
        lane = jax.lax.broadcasted_iota(jnp.int32, (1, 1, 128), 2)

        @pl.when(nvalid == G)
        def _():
            sq = jnp.sum(diff * diff)
            l_ref[...] = jnp.where(lane == 0, sq, 0.0)

        @pl.when(nvalid < G)
        def _():
            toks = jax.lax.broadcasted_iota(jnp.int32, (G, S, D), 0)
            sq = jnp.sum(jnp.where(toks < nvalid, diff * diff, 0.0))
            l_ref[...] = jnp.where(lane == 0, sq, 0.0)


def _run_fused(feats_t, Ws, anchor_tiled, order, starts, rstarts, counts, bd,
               interpret=False):
    grid_spec = pltpu.PrefetchScalarGridSpec(
        num_scalar_prefetch=5,
        grid=(NBLK + 2,),
        in_specs=[
            pl.BlockSpec(memory_space=pl.ANY),
            pl.BlockSpec((1, D, D),
                         lambda g, o, st, rst, cnt, bd: (bd[jnp.minimum(g, NBLK - 1)], 0, 0)),
            pl.BlockSpec((G, S, D), lambda g, o, st, rst, cnt, bd: (0, 0, 0)),
        ],
        out_specs=[
            pl.BlockSpec(memory_space=pl.ANY),
            pl.BlockSpec((1, 1, 128),
                         lambda g, o, st, rst, cnt, bd:
                         (jnp.minimum(jnp.maximum(g - 1, 0), NBLK - 1), 0, 0)),
        ],
        scratch_shapes=[
            pltpu.VMEM((2, G, S, D), jnp.float32),
            pltpu.VMEM((2, G, S, D), jnp.float32),
            pltpu.VMEM((D, D), jnp.bfloat16),
            pltpu.SMEM((2, G), jnp.int32),
            pltpu.SemaphoreType.DMA((2, G)),
            pltpu.SemaphoreType.DMA((2, G)),
        ],
    )
    return pl.pallas_call(
        _fused_body,
        grid_spec=grid_spec,
        out_shape=[
            jax.ShapeDtypeStruct((S, B, D), jnp.float32),
            jax.ShapeDtypeStruct((NBLK, 1, 128), jnp.float32),
        ],
        interpret=interpret,
    )(order, starts, rstarts, counts, bd, feats_t, Ws, anchor_tiled)


def kernel(features, domain_ids, anchor, Ws):
    ids = domain_ids.astype(jnp.int32)
    order, starts, rstarts, counts, bd = _route(ids)
    anchor_tiled = jnp.broadcast_to(anchor.reshape(1, S, D), (G, S, D))
    feats_t = jnp.transpose(features, (1, 0, 2))

    out_t, loss_part = _run_fused(
        feats_t, Ws, anchor_tiled, order, starts, rstarts, counts, bd)
    projected = jnp.transpose(out_t, (1, 0, 2))

    bd_onehot = (bd[:, None] == jnp.arange(ND, dtype=jnp.int32)[None, :]).astype(jnp.float32)
    sq_dom = jnp.sum(loss_part[:, 0, 0][:, None] * bd_onehot, axis=0)
    denom = (jnp.maximum(counts, 1) * S * D).astype(jnp.float32)
    loss = jnp.sum(jnp.where(counts > 0, sq_dom / denom, 0.0)) / ND
    return projected, loss


# speedup vs baseline: 1.0308x; 1.0308x over previous
import jax
import jax.numpy as jnp
from jax.experimental import pallas as pl
from jax.experimental.pallas import tpu as pltpu

ND = 4
D = 1024
B = 1024
S = 20
G = 32
PAD = B + ND * G
NBLK = PAD // G


def _route(ids):
    order = jnp.argsort(ids, stable=True).astype(jnp.int32)
    onehot = (ids[:, None] == jnp.arange(ND, dtype=ids.dtype)[None, :]).astype(jnp.int32)
    counts = jnp.sum(onehot, axis=0)
    rstarts = jnp.cumsum(counts) - counts
    padded = ((counts + G - 1) // G) * G
    ends = jnp.cumsum(padded)
    starts = ends - padded

    gs = jnp.arange(NBLK, dtype=jnp.int32) * G
    bdr = jnp.minimum(
        jnp.sum((gs[:, None] >= ends[None, :]).astype(jnp.int32), axis=1), ND - 1)
    d0 = ids[order[0]].astype(jnp.int32)
    bd = jnp.where(gs < ends[ND - 1], bdr, d0)
    return order, starts, rstarts, counts, bd


def _fused_body(order_ref, st_ref, rst_ref, cnt_ref, bd_ref,
                feat_ref, w_ref, a_ref, out_ref, l_ref,
                xacc, racc, wb, bsave, insem, outsem):
    g = pl.program_id(0)

    def issue_in(gg):
        pp = jax.lax.rem(gg, 2)
        bdv = bd_ref[gg]
        base = gg * G - st_ref[bdv]
        cntv = cnt_ref[bdv]
        rstv = rst_ref[bdv]
        for t in range(G):
            q = base + t
            b = order_ref[rstv + jnp.where(q < cntv, q, 0)]
            bsave[pp, t] = b
            pltpu.make_async_copy(
                feat_ref.at[:, b, :], xacc.at[pp, t], insem.at[pp, t]
            ).start()

    def wait_in(pp):
        for t in range(G):
            pltpu.make_async_copy(
                feat_ref.at[:, 0, :], xacc.at[pp, t], insem.at[pp, t]
            ).wait()

    def issue_out(gg):
        pp = jax.lax.rem(gg, 2)
        for t in range(G):
            b = bsave[pp, t]
            pltpu.make_async_copy(
                racc.at[pp, t], out_ref.at[:, b, :], outsem.at[pp, t]
            ).start()

    def wait_out(pp):
        for t in range(G):
            pltpu.make_async_copy(
                racc.at[pp, t], out_ref.at[:, 0, :], outsem.at[pp, t]
            ).wait()

    @pl.when(g == 0)
    def _():
        issue_in(jnp.int32(0))

    @pl.when(g + 1 < NBLK)
    def _():
        issue_in(g + 1)

    @pl.when(g >= 2)
    def _():
        wait_out(jax.lax.rem(g, 2))

    @pl.when(g < NBLK)
    def _():
        p = jax.lax.rem(g, 2)
        wait_in(p)

        @pl.when((g == 0) | (bd_ref[jnp.maximum(g - 1, 0)] != bd_ref[g]))
        def _():
            wb[...] = w_ref[0].astype(jnp.bfloat16)

        x = xacc[p].astype(jnp.bfloat16)
        w = wb[...]
        res = jax.lax.dot_general(x, w, dimension_numbers=(((2,), (1,)), ((), ())),
                                  preferred_element_type=jnp.float32)
        racc[p] = res
        bdv = bd_ref[g]
        nvalid = jnp.clip(cnt_ref[bdv] - (g * G - st_ref[bdv]), 0, G)
        diff = res - a_ref[...]
        lane = jax.lax.broadcasted_iota(jnp.int32, (1, 1, 128), 2)

        @pl.when(nvalid == G)
        def _():
            sq = jnp.sum(diff * diff)
            l_ref[...] = jnp.where(lane == 0, sq, 0.0)

        @pl.when(nvalid < G)
        def _():
            toks = jax.lax.broadcasted_iota(jnp.int32, (G, S, D), 0)
            sq = jnp.sum(jnp.where(toks < nvalid, diff * diff, 0.0))
            l_ref[...] = jnp.where(lane == 0, sq, 0.0)

        issue_out(g)


def _run_fused(feats_t, Ws, anchor_tiled, order, starts, rstarts, counts, bd,
               interpret=False):
    grid_spec = pltpu.PrefetchScalarGridSpec(
        num_scalar_prefetch=5,
        grid=(NBLK + 2,),
        in_specs=[
            pl.BlockSpec(memory_space=pl.ANY),
            pl.BlockSpec((1, D, D),
                         lambda g, o, st, rst, cnt, bd: (bd[jnp.minimum(g, NBLK - 1)], 0, 0)),
            pl.BlockSpec((G, S, D), lambda g, o, st, rst, cnt, bd: (0, 0, 0)),
        ],
        out_specs=[
            pl.BlockSpec(memory_space=pl.ANY),
            pl.BlockSpec((1, 1, 128),
                         lambda g, o, st, rst, cnt, bd: (jnp.minimum(g, NBLK - 1), 0, 0)),
        ],
        scratch_shapes=[
            pltpu.VMEM((2, G, S, D), jnp.float32),
            pltpu.VMEM((2, G, S, D), jnp.float32),
            pltpu.VMEM((D, D), jnp.bfloat16),
            pltpu.SMEM((2, G), jnp.int32),
            pltpu.SemaphoreType.DMA((2, G)),
            pltpu.SemaphoreType.DMA((2, G)),
        ],
    )
    return pl.pallas_call(
        _fused_body,
        grid_spec=grid_spec,
        out_shape=[
            jax.ShapeDtypeStruct((S, B, D), jnp.float32),
            jax.ShapeDtypeStruct((NBLK, 1, 128), jnp.float32),
        ],
        interpret=interpret,
    )(order, starts, rstarts, counts, bd, feats_t, Ws, anchor_tiled)


def kernel(features, domain_ids, anchor, Ws):
    ids = domain_ids.astype(jnp.int32)
    order, starts, rstarts, counts, bd = _route(ids)
    anchor_tiled = jnp.broadcast_to(anchor.reshape(1, S, D), (G, S, D))
    feats_t = jnp.transpose(features, (1, 0, 2))

    out_t, loss_part = _run_fused(
        feats_t, Ws, anchor_tiled, order, starts, rstarts, counts, bd)
    projected = jnp.transpose(out_t, (1, 0, 2))

    bd_onehot = (bd[:, None] == jnp.arange(ND, dtype=jnp.int32)[None, :]).astype(jnp.float32)
    sq_dom = jnp.sum(loss_part[:, 0, 0][:, None] * bd_onehot, axis=0)
    denom = (jnp.maximum(counts, 1) * S * D).astype(jnp.float32)
    loss = jnp.sum(jnp.where(counts > 0, sq_dom / denom, 0.0)) / ND
    return projected, loss
